# baseline (speedup 1.0000x reference)
import jax
import jax.numpy as jnp
from jax import lax
from jax.experimental import pallas as pl
from jax.experimental.pallas import tpu as pltpu
from jax.experimental.pallas import tpu_sc as plsc

BS = 64
VOCAB = 100000
NC = 128
NRA = 780
ALIGNED = NRA * NC
NRC = 784
GCH = 112
BB = 8
NEG_INF = float("-inf")
BIG = 2 ** 30


def _sweep_select_body(x_ref, tail_ref, scores_ref, wb_ref, wc_ref, wi_ref,
                       scr):
    b = pl.program_id(0)

    def fold(k, m):
        off = pl.multiple_of(k * NC, NC)
        return jnp.maximum(m, x_ref[:, 0, pl.ds(off, NC)])

    m = lax.fori_loop(1, NRA, fold, x_ref[:, 0, pl.ds(0, NC)])
    t = tail_ref[...]
    colmax = jnp.maximum(jnp.maximum(m, t[:, 0:128]), t[:, 128:256])
    scr[pl.ds(b * BB, BB), :] = colmax

    @pl.when(b == BS // BB - 1)
    def _select():
        scr[...] = jnp.log(scr[...]) + jnp.transpose(scores_ref[...])
        flat = (lax.broadcasted_iota(jnp.int32, (BS, NC), 0) * NC
                + lax.broadcasted_iota(jnp.int32, (BS, NC), 1))
        lane = lax.broadcasted_iota(jnp.int32, (1, BS), 1)

        def body(t_, carry):
            wb, wc = carry
            k = scr[...]
            m_ = jnp.max(k)
            cand = jnp.where(k == m_, flat, BIG)
            am = jnp.min(cand)
            scr[...] = jnp.where(flat == am, NEG_INF, k)
            wb = jnp.where(lane == t_, am // NC, wb)
            wc = jnp.where(lane == t_, am % NC, wc)
            return wb, wc

        z = jnp.zeros((1, BS), jnp.int32)
        wb, wc = lax.fori_loop(0, BS, body, (z, z))
        wb_ref[...] = jnp.broadcast_to(wb, (8, BS))
        wc_ref[...] = jnp.broadcast_to(wc, (8, BS))
        r_iota = lax.broadcasted_iota(jnp.int32, (BS, NRC), 1)
        vi = jnp.minimum(jnp.transpose(wc) + NC * r_iota, VOCAB - 1)
        wi_ref[...] = jnp.transpose(wb) * VOCAB + vi


def _sc_gather_body(pflat_ref, idx_ref, cand_ref, idxv, rowsv, sem):
    wid = lax.axis_index("s") * 2 + lax.axis_index("c")
    for k in range(2):
        s = wid * 2 + k
        for g in range(NRC // GCH):
            off = s * NRC + g * GCH
            pltpu.sync_copy(idx_ref.at[pl.ds(off, GCH)], idxv)
            pltpu.async_copy(pflat_ref.at[idxv], rowsv, sem).wait()
            pltpu.sync_copy(rowsv, cand_ref.at[pl.ds(off, GCH)])


def _final_select(cand_ref, scores_ref, wb_ref, wc_ref, ws_ref, wq_ref, wv_ref):
    wb = jnp.transpose(wb_ref[0:1, :])
    wc = jnp.transpose(wc_ref[0:1, :])
    beam_iota = lax.broadcasted_iota(jnp.int32, (BS, BS), 1)
    sc = jnp.broadcast_to(scores_ref[...], (BS, BS))
    sg = jnp.sum(jnp.where(beam_iota == wb, sc, 0.0), axis=1, keepdims=True)
    r_iota = lax.broadcasted_iota(jnp.int32, (BS, NRC), 1)
    v = wc + NC * r_iota
    valid = v < VOCAB
    pmat = cand_ref[...]
    mm = jnp.where(valid, jnp.log(pmat) + sg, NEG_INF)
    lane = lax.broadcasted_iota(jnp.int32, (1, BS), 1)

    def body(t, carry):
        mm, ws, wq, wv = carry
        m = jnp.max(mm)
        sel = mm == m
        bmin = jnp.min(jnp.where(sel, wb, BIG))
        sel = sel & (wb == bmin)
        pmax = jnp.max(jnp.where(sel, pmat, -1.0))
        sel = sel & (pmat == pmax)
        vmin = jnp.min(jnp.where(sel, v, BIG))
        mm = jnp.where(sel & (v == vmin), NEG_INF, mm)
        ws = jnp.where(lane == t, m, ws)
        wq = jnp.where(lane == t, bmin, wq)
        wv = jnp.where(lane == t, vmin, wv)
        return mm, ws, wq, wv

    z = jnp.zeros((1, BS), jnp.int32)
    mm, ws, wq, wv = lax.fori_loop(
        0, BS, body, (mm, jnp.zeros((1, BS), jnp.float32), z, z))
    ws_ref[...] = jnp.broadcast_to(ws, (8, BS))
    wq_ref[...] = jnp.broadcast_to(wq, (8, BS))
    wv_ref[...] = jnp.broadcast_to(wv, (8, BS))


def _gen_body(q_sref, kv_sref, i_sref, grow_ref, gself_ref, o_ref):
    s = pl.program_id(0)
    iv = i_sref[0]
    col = lax.broadcasted_iota(jnp.int32, grow_ref.shape, 2)
    res = jnp.where(col < iv, grow_ref[...], gself_ref[...])
    o_ref[...] = jnp.where(col == iv, kv_sref[s], res)


def kernel(out, scores, gen, i):
    gen = gen.astype(jnp.int32)
    gen_len = gen.shape[1]
    p2 = out.reshape(BS, VOCAB)
    p3 = p2.reshape(BS, 1, VOCAB)
    tail = jnp.pad(p2[:, ALIGNED:], ((0, 0), (0, 256 - (VOCAB - ALIGNED))))

    scores2 = scores.reshape(1, BS)
    wb, wc, wi = pl.pallas_call(
        _sweep_select_body,
        grid=(BS // BB,),
        in_specs=[pl.BlockSpec((BB, 1, ALIGNED), lambda b: (b, 0, 0)),
                  pl.BlockSpec((BB, 256), lambda b: (b, 0)),
                  pl.BlockSpec((1, BS), lambda b: (0, 0))],
        out_specs=[pl.BlockSpec((8, BS), lambda b: (0, 0)),
                   pl.BlockSpec((8, BS), lambda b: (0, 0)),
                   pl.BlockSpec((BS, NRC), lambda b: (0, 0))],
        out_shape=[jax.ShapeDtypeStruct((8, BS), jnp.int32),
                   jax.ShapeDtypeStruct((8, BS), jnp.int32),
                   jax.ShapeDtypeStruct((BS, NRC), jnp.int32)],
        scratch_shapes=[pltpu.VMEM((BS, NC), jnp.float32)],
    )(p3, tail, scores2)

    mesh = plsc.VectorSubcoreMesh(core_axis_name="c", subcore_axis_name="s")
    cand = pl.kernel(
        _sc_gather_body,
        mesh=mesh,
        out_type=jax.ShapeDtypeStruct((BS * NRC,), jnp.float32),
        scratch_types=[
            pltpu.VMEM((GCH,), jnp.int32),
            pltpu.VMEM((GCH,), jnp.float32),
            pltpu.SemaphoreType.DMA,
        ],
    )(p2.reshape(BS * VOCAB), wi.reshape(BS * NRC))
    cand = cand.reshape(BS, NRC)

    ws, wq, wv = pl.pallas_call(
        _final_select,
        out_shape=[jax.ShapeDtypeStruct((8, BS), jnp.float32),
                   jax.ShapeDtypeStruct((8, BS), jnp.int32),
                   jax.ShapeDtypeStruct((8, BS), jnp.int32)],
    )(cand, scores2, wb, wc)

    i_arr = jnp.asarray(i, jnp.int32).reshape(1)
    gen3 = gen.reshape(BS, 1, gen_len)
    gen_new = pl.pallas_call(
        _gen_body,
        grid_spec=pltpu.PrefetchScalarGridSpec(
            num_scalar_prefetch=3,
            grid=(BS,),
            in_specs=[pl.BlockSpec((1, 1, gen_len),
                                   lambda s, q, kv, iv: (q[s], 0, 0)),
                      pl.BlockSpec((1, 1, gen_len),
                                   lambda s, q, kv, iv: (s, 0, 0))],
            out_specs=pl.BlockSpec((1, 1, gen_len),
                                   lambda s, q, kv, iv: (s, 0, 0)),
        ),
        out_shape=jax.ShapeDtypeStruct((BS, 1, gen_len), jnp.int32),
    )(wq[0], wv[0], i_arr, gen3, gen3).reshape(BS, gen_len)

    return gen_new, ws[0].astype(jnp.float32)


# speedup vs baseline: 1.9619x; 1.9619x over previous
import jax
import jax.numpy as jnp
from jax import lax
from jax.experimental import pallas as pl
from jax.experimental.pallas import tpu as pltpu

BS = 64
VOCAB = 100000
NR = 800
NC = 128
PAD = NR * NC - VOCAB
BB = 8
NEG_INF = float("-inf")
BIG = 2 ** 30


def _sweep_select_body(x_ref, scores_ref, wb_ref, wc_ref, scr):
    b = pl.program_id(0)
    scr[pl.ds(b * BB, BB), :] = jnp.max(x_ref[...], axis=1)

    @pl.when(b == BS // BB - 1)
    def _select():
        scr[...] = jnp.log(scr[...]) + jnp.transpose(scores_ref[...])
        flat = (lax.broadcasted_iota(jnp.int32, (BS, NC), 0) * NC
                + lax.broadcasted_iota(jnp.int32, (BS, NC), 1))
        lane = lax.broadcasted_iota(jnp.int32, (1, BS), 1)

        def body(t, carry):
            wb, wc = carry
            k = scr[...]
            m = jnp.max(k)
            cand = jnp.where(k == m, flat, BIG)
            am = jnp.min(cand)
            scr[...] = jnp.where(flat == am, NEG_INF, k)
            wb = jnp.where(lane == t, am // NC, wb)
            wc = jnp.where(lane == t, am % NC, wc)
            return wb, wc

        z = jnp.zeros((1, BS), jnp.int32)
        wb, wc = lax.fori_loop(0, BS, body, (z, z))
        wb_ref[...] = jnp.broadcast_to(wb, (8, BS))
        wc_ref[...] = jnp.broadcast_to(wc, (8, BS))


def _gather_select_body(wb_sref, wc_sref, x_ref, scores_ref, wb8_ref, wc8_ref,
                        ws_ref, wq_ref, wv_ref, cand_scr):
    s = pl.program_id(0)
    c = wc_sref[s]
    sel = lax.broadcasted_iota(jnp.int32, (NR, NC), 1) == c
    col = jnp.max(jnp.where(sel, x_ref[0], 0.0), axis=1)
    cand_scr[pl.ds(s, 1), :] = col.reshape(1, NR)

    @pl.when(s == BS - 1)
    def _final():
        _final_select(cand_scr, scores_ref, wb8_ref, wc8_ref,
                      ws_ref, wq_ref, wv_ref)


def _final_select(cand_ref, scores_ref, wb_ref, wc_ref, ws_ref, wq_ref, wv_ref):
    wb = jnp.transpose(wb_ref[0:1, :])
    wc = jnp.transpose(wc_ref[0:1, :])
    beam_iota = lax.broadcasted_iota(jnp.int32, (BS, BS), 1)
    sc = jnp.broadcast_to(scores_ref[...], (BS, BS))
    sg = jnp.sum(jnp.where(beam_iota == wb, sc, 0.0), axis=1, keepdims=True)
    r_iota = lax.broadcasted_iota(jnp.int32, (BS, NR), 1)
    v = wc + NC * r_iota
    valid = v < VOCAB
    pmat = cand_ref[...]
    mm = jnp.where(valid, jnp.log(pmat) + sg, NEG_INF)
    lane = lax.broadcasted_iota(jnp.int32, (1, BS), 1)

    def body(t, carry):
        mm, ws, wq, wv = carry
        m = jnp.max(mm)
        sel = mm == m
        bmin = jnp.min(jnp.where(sel, wb, BIG))
        sel = sel & (wb == bmin)
        pmax = jnp.max(jnp.where(sel, pmat, -1.0))
        sel = sel & (pmat == pmax)
        vmin = jnp.min(jnp.where(sel, v, BIG))
        mm = jnp.where(sel & (v == vmin), NEG_INF, mm)
        ws = jnp.where(lane == t, m, ws)
        wq = jnp.where(lane == t, bmin, wq)
        wv = jnp.where(lane == t, vmin, wv)
        return mm, ws, wq, wv

    z = jnp.zeros((1, BS), jnp.int32)
    mm, ws, wq, wv = lax.fori_loop(
        0, BS, body, (mm, jnp.zeros((1, BS), jnp.float32), z, z))
    ws_ref[...] = jnp.broadcast_to(ws, (8, BS))
    wq_ref[...] = jnp.broadcast_to(wq, (8, BS))
    wv_ref[...] = jnp.broadcast_to(wv, (8, BS))


def _gen_body(q_sref, kv_sref, i_sref, grow_ref, gself_ref, o_ref):
    s = pl.program_id(0)
    iv = i_sref[0]
    col = lax.broadcasted_iota(jnp.int32, grow_ref.shape, 2)
    res = jnp.where(col < iv, grow_ref[...], gself_ref[...])
    o_ref[...] = jnp.where(col == iv, kv_sref[s], res)


def kernel(out, scores, gen, i):
    gen = gen.astype(jnp.int32)
    gen_len = gen.shape[1]
    p = jnp.pad(out.reshape(BS, VOCAB), ((0, 0), (0, PAD))).reshape(BS, NR, NC)

    scores2 = scores.reshape(1, BS)
    wb, wc = pl.pallas_call(
        _sweep_select_body,
        grid=(BS // BB,),
        in_specs=[pl.BlockSpec((BB, NR, NC), lambda b: (b, 0, 0)),
                  pl.BlockSpec((1, BS), lambda b: (0, 0))],
        out_specs=[pl.BlockSpec((8, BS), lambda b: (0, 0)),
                   pl.BlockSpec((8, BS), lambda b: (0, 0))],
        out_shape=[jax.ShapeDtypeStruct((8, BS), jnp.int32),
                   jax.ShapeDtypeStruct((8, BS), jnp.int32)],
        scratch_shapes=[pltpu.VMEM((BS, NC), jnp.float32)],
    )(p, scores2)

    ws, wq, wv = pl.pallas_call(
        _gather_select_body,
        grid_spec=pltpu.PrefetchScalarGridSpec(
            num_scalar_prefetch=2,
            grid=(BS,),
            in_specs=[pl.BlockSpec((1, NR, NC),
                                   lambda s, wbr, wcr: (wbr[s], 0, 0)),
                      pl.BlockSpec((1, BS), lambda s, wbr, wcr: (0, 0)),
                      pl.BlockSpec((8, BS), lambda s, wbr, wcr: (0, 0)),
                      pl.BlockSpec((8, BS), lambda s, wbr, wcr: (0, 0))],
            out_specs=[pl.BlockSpec((8, BS), lambda s, wbr, wcr: (0, 0)),
                       pl.BlockSpec((8, BS), lambda s, wbr, wcr: (0, 0)),
                       pl.BlockSpec((8, BS), lambda s, wbr, wcr: (0, 0))],
            scratch_shapes=[pltpu.VMEM((BS, NR), jnp.float32)],
        ),
        out_shape=[jax.ShapeDtypeStruct((8, BS), jnp.float32),
                   jax.ShapeDtypeStruct((8, BS), jnp.int32),
                   jax.ShapeDtypeStruct((8, BS), jnp.int32)],
    )(wb[0], wc[0], p, scores2, wb, wc)

    i_arr = jnp.asarray(i, jnp.int32).reshape(1)
    gen3 = gen.reshape(BS, 1, gen_len)
    gen_new = pl.pallas_call(
        _gen_body,
        grid_spec=pltpu.PrefetchScalarGridSpec(
            num_scalar_prefetch=3,
            grid=(BS,),
            in_specs=[pl.BlockSpec((1, 1, gen_len),
                                   lambda s, q, kv, iv: (q[s], 0, 0)),
                      pl.BlockSpec((1, 1, gen_len),
                                   lambda s, q, kv, iv: (s, 0, 0))],
            out_specs=pl.BlockSpec((1, 1, gen_len),
                                   lambda s, q, kv, iv: (s, 0, 0)),
        ),
        out_shape=jax.ShapeDtypeStruct((BS, 1, gen_len), jnp.int32),
    )(wq[0], wv[0], i_arr, gen3, gen3).reshape(BS, gen_len)

    return gen_new, ws[0].astype(jnp.float32)
